# Initial kernel scaffold; baseline (speedup 1.0000x reference)
import jax
import jax.numpy as jnp
from jax.experimental import pallas as pl


def kernel(z, codebook):
    raise NotImplementedError("write your pallas kernel here")


# speedup vs baseline: 1.4215x; 1.4215x over previous
import functools

import jax
import jax.numpy as jnp
from jax import lax
from jax.experimental import pallas as pl
from jax.experimental.pallas import tpu as pltpu
from jax.experimental.pallas import tpu_sc as plsc

K = 8192
D = 256
M = 8192
EPS = 1e-12

BM = 1024
BN = 2048
NB = K // BN
MB = M // BM
QN = 4096 // BN - 1


def _normalize_body(z_ref, cb_ref, zbf_ref, cbf_ref, cbn_ref):
    z = z_ref[...]
    zn = z / jnp.sqrt(jnp.sum(z * z, axis=-1, keepdims=True) + EPS)
    zbf_ref[...] = zn.astype(jnp.bfloat16)
    cb = cb_ref[...]
    cbn = cb / jnp.sqrt(jnp.sum(cb * cb, axis=-1, keepdims=True) + EPS)
    cbn_ref[...] = cbn
    cbf_ref[...] = cbn.astype(jnp.bfloat16)


def _normalize_call(z2d, codebook):
    return pl.pallas_call(
        _normalize_body,
        grid=(MB,),
        in_specs=[
            pl.BlockSpec((BM, D), lambda m: (m, 0)),
            pl.BlockSpec((BM, D), lambda m: (m, 0)),
        ],
        out_specs=[
            pl.BlockSpec((BM, D), lambda m: (m, 0)),
            pl.BlockSpec((BM, D), lambda m: (m, 0)),
            pl.BlockSpec((BM, D), lambda m: (m, 0)),
        ],
        out_shape=[
            jax.ShapeDtypeStruct((M, D), jnp.bfloat16),
            jax.ShapeDtypeStruct((K, D), jnp.bfloat16),
            jax.ShapeDtypeStruct((K, D), jnp.float32),
        ],
    )(z2d, codebook)


def _vq_argmax_body(zbf_ref, cbf_ref, idx_ref, loss_ref, maxv, maxi):
    n = pl.program_id(0)
    m = pl.program_id(1)

    sim = lax.dot_general(cbf_ref[...], zbf_ref[...],
                          (((1,), (1,)), ((), ())),
                          preferred_element_type=jnp.float32)

    bmax = jnp.max(sim, axis=0)
    ids = lax.broadcasted_iota(jnp.int32, (BN, BM), 0)
    bidx = jnp.min(jnp.where(sim == bmax[None, :], ids, jnp.int32(2**31 - 1)),
                   axis=0) + n * BN

    sl = pl.ds(m * BM, BM)

    @pl.when(n == 0)
    def _():
        maxv[sl] = bmax
        maxi[sl] = bidx

    @pl.when(n > 0)
    def _():
        pv = maxv[sl]
        pi = maxi[sl]
        better = bmax > pv
        maxv[sl] = jnp.where(better, bmax, pv)
        maxi[sl] = jnp.where(better, bidx, pi)

    @pl.when(n == QN)
    def _():
        maxv[sl] = maxv[sl].astype(jnp.bfloat16).astype(jnp.float32)

    @pl.when(n == NB - 1)
    def _():
        fv = maxv[sl]
        idx_ref[sl] = maxi[sl]
        partial = jnp.sum(1.0 - fv)

        @pl.when(m == 0)
        def _():
            loss_ref[0, 0] = 0.0

        loss_ref[0, 0] += partial * (2.0 / (M * D))


def _vq_argmax_call(zbf, cbf):
    return pl.pallas_call(
        _vq_argmax_body,
        grid=(NB, MB),
        in_specs=[
            pl.BlockSpec((BM, D), lambda n, m: (m, 0)),
            pl.BlockSpec((BN, D), lambda n, m: (n, 0)),
        ],
        out_specs=[
            pl.BlockSpec((M,), lambda n, m: (0,)),
            pl.BlockSpec(memory_space=pltpu.SMEM),
        ],
        out_shape=[
            jax.ShapeDtypeStruct((M,), jnp.int32),
            jax.ShapeDtypeStruct((1, 1), jnp.float32),
        ],
        scratch_shapes=[
            pltpu.VMEM((M,), jnp.float32),
            pltpu.VMEM((M,), jnp.int32),
        ],
    )(zbf, cbf)


def _make_gather():
    info = plsc.get_sparse_core_info()
    nc, ns = info.num_cores, info.num_subcores
    nw = nc * ns
    bpw = M // nw
    chunks = bpw // 128
    mesh = plsc.VectorSubcoreMesh(core_axis_name="c", subcore_axis_name="s")

    @functools.partial(
        pl.kernel, mesh=mesh,
        out_type=jax.ShapeDtypeStruct((M, D), jnp.float32),
        scratch_types=[
            pltpu.VMEM((chunks, 128), jnp.int32),
            pltpu.VMEM((bpw, D), jnp.float32),
            pltpu.SemaphoreType.DMA,
        ],
    )
    def gather(table_hbm, idx_hbm, out_hbm, idx_v, rows_v, sem):
        wid = lax.axis_index("s") * nc + lax.axis_index("c")
        pltpu.sync_copy(idx_hbm.at[pl.ds(wid * chunks, chunks)], idx_v)
        cps = [
            pltpu.async_copy(table_hbm.at[idx_v.at[j]],
                             rows_v.at[pl.ds(j * 128, 128)], sem)
            for j in range(chunks)
        ]
        for cp in cps:
            cp.wait()
        pltpu.sync_copy(rows_v, out_hbm.at[pl.ds(wid * bpw, bpw)])

    return gather, chunks


def kernel(z, codebook):
    b, t, d = z.shape
    z2d = z.reshape(b * t, d)
    zbf, cbf, cbn = _normalize_call(z2d, codebook)
    idx_flat, loss = _vq_argmax_call(zbf, cbf)
    gather, chunks = _make_gather()
    idx2d = idx_flat.reshape((M // 128), 128)
    quant = gather(cbn, idx2d)
    return (quant.reshape(b, t, d), idx_flat.reshape(b, t), loss[0, 0])


# speedup vs baseline: 1.5108x; 1.0628x over previous
import functools

import jax
import jax.numpy as jnp
from jax import lax
from jax.experimental import pallas as pl
from jax.experimental.pallas import tpu as pltpu
from jax.experimental.pallas import tpu_sc as plsc

K = 8192
D = 256
M = 8192
EPS = 1e-12

BM = 1024
BN = 4096
NB = K // BN
MB = M // BM
QN = 4096 // BN - 1


def _normalize_body(z_ref, cb_ref, zbf_ref, cbf_ref, cbn_ref):
    z = z_ref[...]
    zn = z / jnp.sqrt(jnp.sum(z * z, axis=-1, keepdims=True) + EPS)
    zbf_ref[...] = zn.astype(jnp.bfloat16)
    cb = cb_ref[...]
    cbn = cb / jnp.sqrt(jnp.sum(cb * cb, axis=-1, keepdims=True) + EPS)
    cbn_ref[...] = cbn
    cbf_ref[...] = cbn.astype(jnp.bfloat16)


def _normalize_call(z2d, codebook):
    return pl.pallas_call(
        _normalize_body,
        grid=(MB,),
        in_specs=[
            pl.BlockSpec((BM, D), lambda m: (m, 0)),
            pl.BlockSpec((BM, D), lambda m: (m, 0)),
        ],
        out_specs=[
            pl.BlockSpec((BM, D), lambda m: (m, 0)),
            pl.BlockSpec((BM, D), lambda m: (m, 0)),
            pl.BlockSpec((BM, D), lambda m: (m, 0)),
        ],
        out_shape=[
            jax.ShapeDtypeStruct((M, D), jnp.bfloat16),
            jax.ShapeDtypeStruct((K, D), jnp.bfloat16),
            jax.ShapeDtypeStruct((K, D), jnp.float32),
        ],
    )(z2d, codebook)


def _vq_argmax_body(zbf_ref, cbf_ref, idx_ref, loss_ref, maxv, maxi):
    n = pl.program_id(0)
    m = pl.program_id(1)

    sim = lax.dot_general(cbf_ref[...], zbf_ref[...],
                          (((1,), (1,)), ((), ())),
                          preferred_element_type=jnp.float32)

    bmax = jnp.max(sim, axis=0)
    ids = lax.broadcasted_iota(jnp.int32, (BN, BM), 0)
    bidx = jnp.min(jnp.where(sim == bmax[None, :], ids, jnp.int32(2**31 - 1)),
                   axis=0) + n * BN

    sl = pl.ds(m * BM, BM)

    @pl.when(n == 0)
    def _():
        maxv[sl] = bmax
        maxi[sl] = bidx

    @pl.when(n > 0)
    def _():
        pv = maxv[sl]
        pi = maxi[sl]
        better = bmax > pv
        maxv[sl] = jnp.where(better, bmax, pv)
        maxi[sl] = jnp.where(better, bidx, pi)

    @pl.when(n == QN)
    def _():
        maxv[sl] = maxv[sl].astype(jnp.bfloat16).astype(jnp.float32)

    @pl.when(n == NB - 1)
    def _():
        fv = maxv[sl]
        idx_ref[sl] = maxi[sl]
        partial = jnp.sum(1.0 - fv)

        @pl.when(m == 0)
        def _():
            loss_ref[0, 0] = 0.0

        loss_ref[0, 0] += partial * (2.0 / (M * D))


def _vq_argmax_call(zbf, cbf):
    return pl.pallas_call(
        _vq_argmax_body,
        grid=(NB, MB),
        in_specs=[
            pl.BlockSpec((BM, D), lambda n, m: (m, 0)),
            pl.BlockSpec((BN, D), lambda n, m: (n, 0)),
        ],
        out_specs=[
            pl.BlockSpec((M,), lambda n, m: (0,)),
            pl.BlockSpec(memory_space=pltpu.SMEM),
        ],
        out_shape=[
            jax.ShapeDtypeStruct((M,), jnp.int32),
            jax.ShapeDtypeStruct((1, 1), jnp.float32),
        ],
        scratch_shapes=[
            pltpu.VMEM((M,), jnp.float32),
            pltpu.VMEM((M,), jnp.int32),
        ],
    )(zbf, cbf)


def _make_gather():
    info = plsc.get_sparse_core_info()
    nc, ns = info.num_cores, info.num_subcores
    nw = nc * ns
    bpw = M // nw
    chunks = bpw // 128
    mesh = plsc.VectorSubcoreMesh(core_axis_name="c", subcore_axis_name="s")

    @functools.partial(
        pl.kernel, mesh=mesh,
        out_type=jax.ShapeDtypeStruct((M, D), jnp.float32),
        scratch_types=[
            pltpu.VMEM((chunks, 128), jnp.int32),
            pltpu.VMEM((bpw, D), jnp.float32),
            pltpu.SemaphoreType.DMA,
        ],
    )
    def gather(table_hbm, idx_hbm, out_hbm, idx_v, rows_v, sem):
        wid = lax.axis_index("s") * nc + lax.axis_index("c")
        pltpu.sync_copy(idx_hbm.at[pl.ds(wid * chunks, chunks)], idx_v)
        cps = [
            pltpu.async_copy(table_hbm.at[idx_v.at[j]],
                             rows_v.at[pl.ds(j * 128, 128)], sem)
            for j in range(chunks)
        ]
        for cp in cps:
            cp.wait()
        pltpu.sync_copy(rows_v, out_hbm.at[pl.ds(wid * bpw, bpw)])

    return gather, chunks


def kernel(z, codebook):
    b, t, d = z.shape
    z2d = z.reshape(b * t, d)
    zbf, cbf, cbn = _normalize_call(z2d, codebook)
    idx_flat, loss = _vq_argmax_call(zbf, cbf)
    gather, chunks = _make_gather()
    idx2d = idx_flat.reshape((M // 128), 128)
    quant = gather(cbn, idx2d)
    return (quant.reshape(b, t, d), idx_flat.reshape(b, t), loss[0, 0])


# speedup vs baseline: 1.5671x; 1.0373x over previous
import functools

import jax
import jax.numpy as jnp
from jax import lax
from jax.experimental import pallas as pl
from jax.experimental.pallas import tpu as pltpu
from jax.experimental.pallas import tpu_sc as plsc

K = 8192
D = 256
M = 8192
EPS = 1e-12

BM = 1024
BN = 4096
NB = K // BN
MB = M // BM
QN = 4096 // BN - 1


def _normalize_body(z_ref, cb_ref, zbf_ref, cbf_ref, cbn_ref):
    z = z_ref[...]
    zn = z / jnp.sqrt(jnp.sum(z * z, axis=-1, keepdims=True) + EPS)
    zbf_ref[...] = zn.astype(jnp.bfloat16)
    cb = cb_ref[...]
    cbn = cb / jnp.sqrt(jnp.sum(cb * cb, axis=-1, keepdims=True) + EPS)
    cbn_ref[...] = cbn
    cbf_ref[...] = cbn.astype(jnp.bfloat16)


def _normalize_call(z2d, codebook):
    return pl.pallas_call(
        _normalize_body,
        grid=(MB,),
        in_specs=[
            pl.BlockSpec((BM, D), lambda m: (m, 0)),
            pl.BlockSpec((BM, D), lambda m: (m, 0)),
        ],
        out_specs=[
            pl.BlockSpec((BM, D), lambda m: (m, 0)),
            pl.BlockSpec((BM, D), lambda m: (m, 0)),
            pl.BlockSpec((BM, D), lambda m: (m, 0)),
        ],
        out_shape=[
            jax.ShapeDtypeStruct((M, D), jnp.bfloat16),
            jax.ShapeDtypeStruct((K, D), jnp.bfloat16),
            jax.ShapeDtypeStruct((K, D), jnp.float32),
        ],
    )(z2d, codebook)


def _vq_argmax_body(zbf_ref, cbf_ref, idx_ref, loss_ref, maxv, maxi):
    n = pl.program_id(0)
    m = pl.program_id(1)

    sim = lax.dot_general(cbf_ref[...], zbf_ref[...],
                          (((1,), (1,)), ((), ())),
                          preferred_element_type=jnp.float32)

    bmax = jnp.max(sim, axis=0)
    ids = lax.broadcasted_iota(jnp.int32, (BN, BM), 0).astype(jnp.float32)
    bidx_f = jnp.min(jnp.where(sim == bmax[None, :], ids, jnp.float32(2**30)),
                     axis=0)
    bidx = bidx_f.astype(jnp.int32) + n * BN

    sl = pl.ds(m * BM, BM)

    @pl.when(n == 0)
    def _():
        maxv[sl] = bmax
        maxi[sl] = bidx

    @pl.when(n > 0)
    def _():
        pv = maxv[sl]
        pi = maxi[sl]
        better = bmax > pv
        maxv[sl] = jnp.where(better, bmax, pv)
        maxi[sl] = jnp.where(better, bidx, pi)

    @pl.when(n == QN)
    def _():
        maxv[sl] = maxv[sl].astype(jnp.bfloat16).astype(jnp.float32)

    @pl.when(n == NB - 1)
    def _():
        fv = maxv[sl]
        idx_ref[sl] = maxi[sl]
        partial = jnp.sum(1.0 - fv)

        @pl.when(m == 0)
        def _():
            loss_ref[0, 0] = 0.0

        loss_ref[0, 0] += partial * (2.0 / (M * D))


def _vq_argmax_call(zbf, cbf):
    return pl.pallas_call(
        _vq_argmax_body,
        grid=(NB, MB),
        in_specs=[
            pl.BlockSpec((BM, D), lambda n, m: (m, 0)),
            pl.BlockSpec((BN, D), lambda n, m: (n, 0)),
        ],
        out_specs=[
            pl.BlockSpec((M,), lambda n, m: (0,)),
            pl.BlockSpec(memory_space=pltpu.SMEM),
        ],
        out_shape=[
            jax.ShapeDtypeStruct((M,), jnp.int32),
            jax.ShapeDtypeStruct((1, 1), jnp.float32),
        ],
        scratch_shapes=[
            pltpu.VMEM((M,), jnp.float32),
            pltpu.VMEM((M,), jnp.int32),
        ],
    )(zbf, cbf)


def _make_gather():
    info = plsc.get_sparse_core_info()
    nc, ns = info.num_cores, info.num_subcores
    nw = nc * ns
    bpw = M // nw
    chunks = bpw // 128
    mesh = plsc.VectorSubcoreMesh(core_axis_name="c", subcore_axis_name="s")

    @functools.partial(
        pl.kernel, mesh=mesh,
        out_type=jax.ShapeDtypeStruct((M, D), jnp.float32),
        scratch_types=[
            pltpu.VMEM((chunks, 128), jnp.int32),
            pltpu.VMEM((bpw, D), jnp.float32),
            pltpu.SemaphoreType.DMA,
        ],
    )
    def gather(table_hbm, idx_hbm, out_hbm, idx_v, rows_v, sem):
        wid = lax.axis_index("s") * nc + lax.axis_index("c")
        pltpu.sync_copy(idx_hbm.at[pl.ds(wid * chunks, chunks)], idx_v)
        cps = [
            pltpu.async_copy(table_hbm.at[idx_v.at[j]],
                             rows_v.at[pl.ds(j * 128, 128)], sem)
            for j in range(chunks)
        ]
        for cp in cps:
            cp.wait()
        pltpu.sync_copy(rows_v, out_hbm.at[pl.ds(wid * bpw, bpw)])

    return gather, chunks


def kernel(z, codebook):
    b, t, d = z.shape
    z2d = z.reshape(b * t, d)
    zbf, cbf, cbn = _normalize_call(z2d, codebook)
    idx_flat, loss = _vq_argmax_call(zbf, cbf)
    gather, chunks = _make_gather()
    idx2d = idx_flat.reshape((M // 128), 128)
    quant = gather(cbn, idx2d)
    return (quant.reshape(b, t, d), idx_flat.reshape(b, t), loss[0, 0])


# speedup vs baseline: 1.5713x; 1.0027x over previous
import functools

import jax
import jax.numpy as jnp
from jax import lax
from jax.experimental import pallas as pl
from jax.experimental.pallas import tpu as pltpu
from jax.experimental.pallas import tpu_sc as plsc

K = 8192
D = 256
M = 8192
EPS = 1e-12

BM = 1024
BN = 4096
NB = K // BN
MB = M // BM
QN = 4096 // BN - 1


def _normalize_body(z_ref, cb_ref, zbf_ref, cbf_ref, cbn_ref):
    z = z_ref[...]
    zn = z / jnp.sqrt(jnp.sum(z * z, axis=-1, keepdims=True) + EPS)
    zbf_ref[...] = zn.astype(jnp.bfloat16)
    cb = cb_ref[...]
    cbn = cb / jnp.sqrt(jnp.sum(cb * cb, axis=-1, keepdims=True) + EPS)
    cbn_ref[...] = cbn
    cbf_ref[...] = cbn.astype(jnp.bfloat16)


def _normalize_call(z2d, codebook):
    return pl.pallas_call(
        _normalize_body,
        grid=(MB,),
        in_specs=[
            pl.BlockSpec((BM, D), lambda m: (m, 0)),
            pl.BlockSpec((BM, D), lambda m: (m, 0)),
        ],
        out_specs=[
            pl.BlockSpec((BM, D), lambda m: (m, 0)),
            pl.BlockSpec((BM, D), lambda m: (m, 0)),
            pl.BlockSpec((BM, D), lambda m: (m, 0)),
        ],
        out_shape=[
            jax.ShapeDtypeStruct((M, D), jnp.bfloat16),
            jax.ShapeDtypeStruct((K, D), jnp.bfloat16),
            jax.ShapeDtypeStruct((K, D), jnp.float32),
        ],
    )(z2d, codebook)


def _vq_argmax_body(zbf_ref, cbf_ref, idx_ref, loss_ref, maxv, maxi):
    n = pl.program_id(0)
    m = pl.program_id(1)

    sim = lax.dot_general(cbf_ref[...], zbf_ref[...],
                          (((1,), (1,)), ((), ())),
                          preferred_element_type=jnp.float32)

    bmax = jnp.max(sim, axis=0)
    ids = lax.broadcasted_iota(jnp.int32, (BN, 1), 0).astype(jnp.float32)
    bidx_f = jnp.min(jnp.where(sim == bmax[None, :], ids, jnp.float32(2**30)),
                     axis=0)
    bidx = bidx_f.astype(jnp.int32) + n * BN

    sl = pl.ds(m * BM, BM)

    @pl.when(n == 0)
    def _():
        maxv[sl] = bmax
        maxi[sl] = bidx

    @pl.when(n > 0)
    def _():
        pv = maxv[sl]
        pi = maxi[sl]
        better = bmax > pv
        maxv[sl] = jnp.where(better, bmax, pv)
        maxi[sl] = jnp.where(better, bidx, pi)

    @pl.when(n == QN)
    def _():
        maxv[sl] = maxv[sl].astype(jnp.bfloat16).astype(jnp.float32)

    @pl.when(n == NB - 1)
    def _():
        fv = maxv[sl]
        idx_ref[sl] = maxi[sl]
        partial = jnp.sum(1.0 - fv)

        @pl.when(m == 0)
        def _():
            loss_ref[0, 0] = 0.0

        loss_ref[0, 0] += partial * (2.0 / (M * D))


def _vq_argmax_call(zbf, cbf):
    return pl.pallas_call(
        _vq_argmax_body,
        grid=(NB, MB),
        in_specs=[
            pl.BlockSpec((BM, D), lambda n, m: (m, 0)),
            pl.BlockSpec((BN, D), lambda n, m: (n, 0)),
        ],
        out_specs=[
            pl.BlockSpec((M,), lambda n, m: (0,)),
            pl.BlockSpec(memory_space=pltpu.SMEM),
        ],
        out_shape=[
            jax.ShapeDtypeStruct((M,), jnp.int32),
            jax.ShapeDtypeStruct((1, 1), jnp.float32),
        ],
        scratch_shapes=[
            pltpu.VMEM((M,), jnp.float32),
            pltpu.VMEM((M,), jnp.int32),
        ],
    )(zbf, cbf)


def _make_gather():
    info = plsc.get_sparse_core_info()
    nc, ns = info.num_cores, info.num_subcores
    nw = nc * ns
    bpw = M // nw
    chunks = bpw // 128
    mesh = plsc.VectorSubcoreMesh(core_axis_name="c", subcore_axis_name="s")

    @functools.partial(
        pl.kernel, mesh=mesh,
        out_type=jax.ShapeDtypeStruct((M, D), jnp.float32),
        scratch_types=[
            pltpu.VMEM((chunks, 128), jnp.int32),
            pltpu.VMEM((bpw, D), jnp.float32),
            pltpu.SemaphoreType.DMA,
        ],
    )
    def gather(table_hbm, idx_hbm, out_hbm, idx_v, rows_v, sem):
        wid = lax.axis_index("s") * nc + lax.axis_index("c")
        pltpu.sync_copy(idx_hbm.at[pl.ds(wid * chunks, chunks)], idx_v)
        cps = [
            pltpu.async_copy(table_hbm.at[idx_v.at[j]],
                             rows_v.at[pl.ds(j * 128, 128)], sem)
            for j in range(chunks)
        ]
        for cp in cps:
            cp.wait()
        pltpu.sync_copy(rows_v, out_hbm.at[pl.ds(wid * bpw, bpw)])

    return gather, chunks


def kernel(z, codebook):
    b, t, d = z.shape
    z2d = z.reshape(b * t, d)
    zbf, cbf, cbn = _normalize_call(z2d, codebook)
    idx_flat, loss = _vq_argmax_call(zbf, cbf)
    gather, chunks = _make_gather()
    idx2d = idx_flat.reshape((M // 128), 128)
    quant = gather(cbn, idx2d)
    return (quant.reshape(b, t, d), idx_flat.reshape(b, t), loss[0, 0])


# speedup vs baseline: 1.7154x; 1.0917x over previous
import functools

import jax
import jax.numpy as jnp
from jax import lax
from jax.experimental import pallas as pl
from jax.experimental.pallas import tpu as pltpu
from jax.experimental.pallas import tpu_sc as plsc

K = 8192
D = 256
M = 8192
EPS = 1e-12

BM = 2048
BN = 4096
NB = K // BN
MB = M // BM
QN = 4096 // BN - 1


def _normalize_body(z_ref, cb_ref, zbf_ref, cbf_ref, cbn_ref):
    z = z_ref[...]
    zn = z / jnp.sqrt(jnp.sum(z * z, axis=-1, keepdims=True) + EPS)
    zbf_ref[...] = zn.astype(jnp.bfloat16)
    cb = cb_ref[...]
    cbn = cb / jnp.sqrt(jnp.sum(cb * cb, axis=-1, keepdims=True) + EPS)
    cbn_ref[...] = cbn
    cbf_ref[...] = cbn.astype(jnp.bfloat16)


def _normalize_call(z2d, codebook):
    return pl.pallas_call(
        _normalize_body,
        grid=(MB,),
        in_specs=[
            pl.BlockSpec((BM, D), lambda m: (m, 0)),
            pl.BlockSpec((BM, D), lambda m: (m, 0)),
        ],
        out_specs=[
            pl.BlockSpec((BM, D), lambda m: (m, 0)),
            pl.BlockSpec((BM, D), lambda m: (m, 0)),
            pl.BlockSpec((BM, D), lambda m: (m, 0)),
        ],
        out_shape=[
            jax.ShapeDtypeStruct((M, D), jnp.bfloat16),
            jax.ShapeDtypeStruct((K, D), jnp.bfloat16),
            jax.ShapeDtypeStruct((K, D), jnp.float32),
        ],
    )(z2d, codebook)


def _vq_argmax_body(zbf_ref, cbf_ref, idx_ref, loss_ref, maxv, maxi):
    n = pl.program_id(0)
    m = pl.program_id(1)

    sim = lax.dot_general(cbf_ref[...], zbf_ref[...],
                          (((1,), (1,)), ((), ())),
                          preferred_element_type=jnp.float32)

    bmax = jnp.max(sim, axis=0)
    ids = lax.broadcasted_iota(jnp.int32, (BN, 1), 0).astype(jnp.float32)
    bidx_f = jnp.min(jnp.where(sim == bmax[None, :], ids, jnp.float32(2**30)),
                     axis=0)
    bidx = bidx_f.astype(jnp.int32) + n * BN

    sl = pl.ds(m * BM, BM)

    @pl.when(n == 0)
    def _():
        maxv[sl] = bmax
        maxi[sl] = bidx

    @pl.when(n > 0)
    def _():
        pv = maxv[sl]
        pi = maxi[sl]
        better = bmax > pv
        maxv[sl] = jnp.where(better, bmax, pv)
        maxi[sl] = jnp.where(better, bidx, pi)

    @pl.when(n == QN)
    def _():
        maxv[sl] = maxv[sl].astype(jnp.bfloat16).astype(jnp.float32)

    @pl.when(n == NB - 1)
    def _():
        fv = maxv[sl]
        idx_ref[sl] = maxi[sl]
        partial = jnp.sum(1.0 - fv)

        @pl.when(m == 0)
        def _():
            loss_ref[0, 0] = 0.0

        loss_ref[0, 0] += partial * (2.0 / (M * D))


def _vq_argmax_call(zbf, cbf):
    return pl.pallas_call(
        _vq_argmax_body,
        grid=(NB, MB),
        in_specs=[
            pl.BlockSpec((BM, D), lambda n, m: (m, 0)),
            pl.BlockSpec((BN, D), lambda n, m: (n, 0)),
        ],
        out_specs=[
            pl.BlockSpec((M,), lambda n, m: (0,)),
            pl.BlockSpec(memory_space=pltpu.SMEM),
        ],
        out_shape=[
            jax.ShapeDtypeStruct((M,), jnp.int32),
            jax.ShapeDtypeStruct((1, 1), jnp.float32),
        ],
        scratch_shapes=[
            pltpu.VMEM((M,), jnp.float32),
            pltpu.VMEM((M,), jnp.int32),
        ],
    )(zbf, cbf)


def _make_gather():
    info = plsc.get_sparse_core_info()
    nc, ns = info.num_cores, info.num_subcores
    nw = nc * ns
    bpw = M // nw
    chunks = bpw // 128
    mesh = plsc.VectorSubcoreMesh(core_axis_name="c", subcore_axis_name="s")

    @functools.partial(
        pl.kernel, mesh=mesh,
        out_type=jax.ShapeDtypeStruct((M, D), jnp.float32),
        scratch_types=[
            pltpu.VMEM((chunks, 128), jnp.int32),
            pltpu.VMEM((bpw, D), jnp.float32),
            pltpu.SemaphoreType.DMA,
        ],
    )
    def gather(table_hbm, idx_hbm, out_hbm, idx_v, rows_v, sem):
        wid = lax.axis_index("s") * nc + lax.axis_index("c")
        pltpu.sync_copy(idx_hbm.at[pl.ds(wid * chunks, chunks)], idx_v)
        cps = [
            pltpu.async_copy(table_hbm.at[idx_v.at[j]],
                             rows_v.at[pl.ds(j * 128, 128)], sem)
            for j in range(chunks)
        ]
        for cp in cps:
            cp.wait()
        pltpu.sync_copy(rows_v, out_hbm.at[pl.ds(wid * bpw, bpw)])

    return gather, chunks


def kernel(z, codebook):
    b, t, d = z.shape
    z2d = z.reshape(b * t, d)
    zbf, cbf, cbn = _normalize_call(z2d, codebook)
    idx_flat, loss = _vq_argmax_call(zbf, cbf)
    gather, chunks = _make_gather()
    idx2d = idx_flat.reshape((M // 128), 128)
    quant = gather(cbn, idx2d)
    return (quant.reshape(b, t, d), idx_flat.reshape(b, t), loss[0, 0])
